# baseline (speedup 1.0000x reference)
import jax
import jax.numpy as jnp
from jax.experimental import pallas as pl
from jax.experimental.pallas import tpu as pltpu

N_ROWS = 100000
N_FEAT = 12
S = 20
G = 5
BLOCK = N_ROWS // (S * G)


def _kern(*refs):
    x_refs = refs[:S]
    wa1_ref, ba1_ref, wa5_ref, ba5_ref, out_ref, acc_ref = refs[S:]
    i = pl.program_id(0)

    @pl.when(i == 0)
    def _init():
        acc_ref[...] = jnp.zeros_like(acc_ref)

    part = x_refs[0][...]
    for k in range(1, S):
        part = part + x_refs[k][...]
    acc_ref[...] += jnp.sum(part, axis=0, keepdims=True)

    @pl.when(i == pl.num_programs(0) - 1)
    def _finish():
        x_prime = acc_ref[...] * (2.0 / N_ROWS)
        a1 = jnp.sum(wa1_ref[...] * x_prime, axis=1, keepdims=True).T
        a1 = jnp.maximum(a1 + ba1_ref[...], 0.0)
        out_ref[...] = (
            jnp.sum(a1 * wa5_ref[...], axis=1, keepdims=True) + ba5_ref[...]
        )


def kernel(x, edge_index, W1_rel, b1_rel, W1_root, W2_rel, b2_rel, W2_root,
           Wa1, ba1, Wa5, ba5):
    del edge_index, W1_rel, b1_rel, W1_root, W2_rel, b2_rel, W2_root
    x_specs = [
        pl.BlockSpec((BLOCK, N_FEAT), lambda i, k=k: (i * S + k, 0))
        for k in range(S)
    ]
    return pl.pallas_call(
        _kern,
        grid=(G,),
        in_specs=x_specs + [
            pl.BlockSpec((11, 12), lambda i: (0, 0)),
            pl.BlockSpec((1, 11), lambda i: (0, 0)),
            pl.BlockSpec((1, 11), lambda i: (0, 0)),
            pl.BlockSpec((1, 1), lambda i: (0, 0)),
        ],
        out_specs=pl.BlockSpec((1, 1), lambda i: (0, 0)),
        out_shape=jax.ShapeDtypeStruct((1, 1), jnp.float32),
        scratch_shapes=[pltpu.VMEM((1, N_FEAT), jnp.float32)],
    )(*([x] * S), Wa1, ba1.reshape(1, 11), Wa5, ba5.reshape(1, 1))
